# baseline (speedup 1.0000x reference)
import functools

import jax
import jax.numpy as jnp
from jax import lax
from jax.experimental import pallas as pl
from jax.experimental.pallas import tpu as pltpu
from jax.experimental.pallas import tpu_sc as plsc

N = 10000
FD = 128
H = 64
K = 32
E = 320000

NC = 2
NS = 16
LANES = 16

ROWS_PER_TILE = N // NS
CHUNK = 80
EDGES_PER_TILE = E // (NC * NS)
NUM_CHUNKS = EDGES_PER_TILE // CHUNK

_mesh = functools.partial(
    plsc.VectorSubcoreMesh, core_axis_name="c", subcore_axis_name="s",
    num_cores=NC, num_subcores=NS)


def _leaky(x):
    return jnp.where(x >= 0, x, 0.2 * x)


def _k1_body(x_ref, wcat_ref, bcat_ref, att1_ref,
             xl_ref, xr_ref, am_ref, bm_ref, s1_ref):
    x = x_ref[...]
    o = jnp.dot(x, wcat_ref[...], preferred_element_type=jnp.float32)
    o = o + bcat_ref[...]
    xl = o[:, :H]
    xr = o[:, H:2 * H]
    xl_ref[...] = xl
    xr_ref[...] = xr
    am_ref[...] = o[:, 2 * H:2 * H + K]
    bm_ref[...] = o[:, 2 * H + K:]
    colmax = jnp.max(jnp.abs(xl), axis=0) + jnp.max(jnp.abs(xr), axis=0)
    s1_ref[...] = jnp.sum(jnp.abs(att1_ref[0, :]) * colmax)[None, None]


def _tc_k1(X, Wcat, bcat, att1):
    return pl.pallas_call(
        _k1_body,
        out_shape=[
            jax.ShapeDtypeStruct((N, H), jnp.float32),
            jax.ShapeDtypeStruct((N, H), jnp.float32),
            jax.ShapeDtypeStruct((N, K), jnp.float32),
            jax.ShapeDtypeStruct((N, K), jnp.float32),
            jax.ShapeDtypeStruct((1, 1), jnp.float32),
        ],
    )(X, Wcat, bcat, att1)


def _k2_body(p_ref, xl1_ref, xr1_ref, att1_ref, bias1_ref, s1_ref,
             w2_ref, b2_ref, att2_ref,
             xl2_ref, xr2_ref, s2_ref):
    num = p_ref[0, :N, :H] + p_ref[1, :N, :H]
    dseg = p_ref[0, :N, H:H + 1] + p_ref[1, :N, H:H + 1]
    xl1 = xl1_ref[...]
    sl = jnp.sum(att1_ref[...] * _leaky(xl1 + xr1_ref[...]), axis=1,
                 keepdims=True)
    e_self = jnp.exp(sl - s1_ref[0, 0])
    den = dseg + e_self
    h = (num + e_self * xl1) / den + bias1_ref[...]
    h = jnp.where(h > 0, h, jnp.exp(jnp.minimum(h, 0.0)) - 1.0)
    hw = jnp.dot(h, w2_ref[...], preferred_element_type=jnp.float32)
    hw = hw + b2_ref[...]
    xl2 = hw[:, :K]
    xr2 = hw[:, K:]
    xl2_ref[...] = xl2
    xr2_ref[...] = xr2
    colmax = jnp.max(jnp.abs(xl2), axis=0) + jnp.max(jnp.abs(xr2), axis=0)
    s2_ref[...] = jnp.sum(jnp.abs(att2_ref[0, :]) * colmax)[None, None]


def _tc_k2(P1, xl1, xr1, att1, bias1, S1, W2cat, b2cat, att2):
    return pl.pallas_call(
        _k2_body,
        out_shape=[
            jax.ShapeDtypeStruct((N, K), jnp.float32),
            jax.ShapeDtypeStruct((N, K), jnp.float32),
            jax.ShapeDtypeStruct((1, 1), jnp.float32),
        ],
    )(P1, xl1, xr1, att1, bias1, S1, W2cat, b2cat, att2)


def _k3a_body(p_ref, xl2_ref, xr2_ref, att2_ref, bias2_ref, s2_ref, z_ref):
    num = p_ref[0, :N, :K] + p_ref[1, :N, :K]
    dseg = p_ref[0, :N, K:K + 1] + p_ref[1, :N, K:K + 1]
    xl2 = xl2_ref[...]
    sl = jnp.sum(att2_ref[...] * _leaky(xl2 + xr2_ref[...]), axis=1,
                 keepdims=True)
    e_self = jnp.exp(sl - s2_ref[0, 0])
    den = dseg + e_self
    o = (num + e_self * xl2) / den + bias2_ref[...]
    m = jnp.max(o, axis=1, keepdims=True)
    z = jnp.exp(o - m)
    z_ref[...] = z / jnp.sum(z, axis=1, keepdims=True)


def _tc_k3a(P2, xl2, xr2, att2, bias2, S2):
    return pl.pallas_call(
        _k3a_body,
        out_shape=jax.ShapeDtypeStruct((N, K), jnp.float32),
    )(P2, xl2, xr2, att2, bias2, S2)


def _k3b_body(z_ref, bz_ref, m_ref, out_ref):
    zt = z_ref[...] + bz_ref[0, :N, :] + bz_ref[1, :N, :]
    mr = jnp.maximum(m_ref[...], 0.0)
    out_ref[...] = jnp.dot(zt, mr, preferred_element_type=jnp.float32)


def _tc_k3b(Z, BZ, M):
    return pl.pallas_call(
        _k3b_body,
        out_shape=jax.ShapeDtypeStruct((N, FD), jnp.float32),
    )(Z, BZ, M)


def _gat_edge_body(D, W, STAGE,
                   xl_hbm, xr_hbm, src_hbm, dst_hbm, attbc_hbm, s_hbm,
                   p_hbm,
                   sidx_v, didx_v, xlr_v, xrr_v, buf_v, attbc_v, s_v,
                   xl_sp, xr_sp, acc_sp,
                   sem1a, sem2a, sem1b, sem2b, semsca, semscb):
    cid = lax.axis_index("c")
    sid = lax.axis_index("s")
    wid = cid * NS + sid

    pltpu.sync_copy(attbc_hbm, attbc_v)
    pltpu.sync_copy(s_hbm, s_v)
    pltpu.sync_copy(src_hbm.at[wid], sidx_v)
    pltpu.sync_copy(dst_hbm.at[wid], didx_v)

    row0 = sid * ROWS_PER_TILE
    if STAGE:
        cps = pltpu.async_copy(xl_hbm.at[pl.ds(row0, ROWS_PER_TILE)],
                               xl_sp.at[pl.ds(row0, ROWS_PER_TILE)], semscb)
        cpr = pltpu.async_copy(xr_hbm.at[pl.ds(row0, ROWS_PER_TILE)],
                               xr_sp.at[pl.ds(row0, ROWS_PER_TILE)], semscb)
    for j in range(LANES):
        for jj in range(W // LANES):
            buf_v[0, j, pl.ds(jj * LANES, LANES)] = (
                jnp.zeros((LANES,), jnp.float32))
    zrows = [j * LANES for j in range(ROWS_PER_TILE // LANES)]
    zrows.append(ROWS_PER_TILE - LANES)
    for r in zrows:
        pltpu.async_copy(buf_v.at[0, pl.ds(0, LANES)],
                         acc_sp.at[pl.ds(row0 + r, LANES)], semsca)
    for r in zrows:
        pltpu.make_async_copy(buf_v.at[0, pl.ds(0, LANES)],
                              acc_sp.at[pl.ds(row0, LANES)], semsca).wait()
    if STAGE:
        cps.wait()
        cpr.wait()
    plsc.subcore_barrier()

    svec = s_v[...]
    att = [attbc_v[pl.ds(j * LANES, LANES)] for j in range(D // LANES)]

    def compute_chunk(t, xl_cur, xr_cur, bbuf, bsem):
        @plsc.parallel_loop(0, CHUNK, 1, unroll=(5 if D > 32 else 8))
        def edge_body(e):
            acc1 = jnp.zeros((LANES,), jnp.float32)
            acc2 = jnp.zeros((LANES,), jnp.float32)
            vls = []
            for j in range(D // LANES):
                vl = xl_cur[e, pl.ds(j * LANES, LANES)]
                vr = xr_cur[e, pl.ds(j * LANES, LANES)]
                vls.append(vl)
                sj = vl + vr
                acc1 = acc1 + sj * att[j]
                acc2 = acc2 + jnp.abs(sj) * att[j]
            tot = 0.6 * acc1 + 0.4 * acc2
            ev = jnp.exp(
                jnp.broadcast_to(jnp.sum(tot, axis=0), (LANES,)) - svec)
            for j in range(D // LANES):
                bbuf[e, pl.ds(j * LANES, LANES)] = vls[j] * ev
            bbuf[e, pl.ds(D, LANES)] = ev

        pltpu.async_copy(bbuf, acc_sp.at[didx_v.at[t]], bsem, add=True)

    bufs = [(xlr_v.at[0], xrr_v.at[0], sem1a, sem2a, buf_v.at[0], semsca),
            (xlr_v.at[1], xrr_v.at[1], sem1b, sem2b, buf_v.at[1], semscb)]

    xl_src = xl_sp if STAGE else xl_hbm
    xr_src = xr_sp if STAGE else xr_hbm

    def issue(t, bs):
        pltpu.async_copy(xl_src.at[sidx_v.at[t]], bs[0], bs[2])
        pltpu.async_copy(xr_src.at[didx_v.at[t]], bs[1], bs[3])

    def wait_for(t, bs):
        pltpu.make_async_copy(xl_src.at[sidx_v.at[t]], bs[0], bs[2]).wait()
        pltpu.make_async_copy(xr_src.at[didx_v.at[t]], bs[1], bs[3]).wait()

    def wait_scatter(t, bs):
        pltpu.make_async_copy(bs[4], acc_sp.at[didx_v.at[t]], bs[5]).wait()

    issue(0, bufs[0])

    def pair_body(u, carry):
        t0 = u * 2
        for b in range(2):
            t = t0 + b
            issue(t + 1, bufs[1 - b])
            wait_for(t, bufs[b])

            @pl.when(t >= 2)
            def _():
                wait_scatter(t, bufs[b])

            compute_chunk(t, bufs[b][0], bufs[b][1], bufs[b][4], bufs[b][5])
        return carry

    lax.fori_loop(0, (NUM_CHUNKS - 1) // 2, pair_body, 0)
    t_last = NUM_CHUNKS - 1
    wait_for(t_last, bufs[0])
    wait_scatter(t_last, bufs[0])
    compute_chunk(t_last, bufs[0][0], bufs[0][1], bufs[0][4], bufs[0][5])
    wait_scatter(t_last, bufs[0])
    wait_scatter(t_last, bufs[1])
    plsc.subcore_barrier()

    pltpu.sync_copy(acc_sp.at[pl.ds(row0, ROWS_PER_TILE)],
                    p_hbm.at[cid, pl.ds(row0, ROWS_PER_TILE)])


def _sc_gat_pass(D, xl, xr, src, dst, attbc, s16):
    W = D + LANES
    stage = D <= 32
    body = functools.partial(_gat_edge_body, D, W, stage)
    kern = pl.kernel(
        body,
        out_type=jax.ShapeDtypeStruct((NC, N, W), jnp.float32),
        mesh=_mesh(),
        compiler_params=pltpu.CompilerParams(needs_layout_passes=False, use_tc_tiling_on_sc=False),
        scratch_types=[
            pltpu.VMEM((NUM_CHUNKS, CHUNK), jnp.int32),
            pltpu.VMEM((NUM_CHUNKS, CHUNK), jnp.int32),
            pltpu.VMEM((2, CHUNK, D), jnp.float32),
            pltpu.VMEM((2, CHUNK, D), jnp.float32),
            pltpu.VMEM((2, CHUNK, W), jnp.float32),
            pltpu.VMEM((D,), jnp.float32),
            pltpu.VMEM((LANES,), jnp.float32),
            pltpu.VMEM_SHARED((N, D) if stage else (8, LANES), jnp.float32),
            pltpu.VMEM_SHARED((N, D) if stage else (8, LANES), jnp.float32),
            pltpu.VMEM_SHARED((N, W), jnp.float32),
            pltpu.SemaphoreType.DMA,
            pltpu.SemaphoreType.DMA,
            pltpu.SemaphoreType.DMA,
            pltpu.SemaphoreType.DMA,
            pltpu.SemaphoreType.DMA,
            pltpu.SemaphoreType.DMA,
        ],
    )
    return kern(xl, xr, src, dst, attbc, s16)


def _mlp_edge_body(am_hbm, bm_hbm, z_hbm, row_hbm, col_hbm, w2bc_hbm, bm2_hbm,
                   w_hbm, bz_hbm,
                   ridx_v, cidx_v, ar_v, br_v, zr_v, buf_v, wrow_v, wbuf_v,
                   w2bc_v, bm2_v, bm_sp, z_sp, acc_sp,
                   sem1a, sem2a, sem3a, sem1b, sem2b, sem3b,
                   semsca, semscb):
    cid = lax.axis_index("c")
    sid = lax.axis_index("s")
    wid = cid * NS + sid

    pltpu.sync_copy(w2bc_hbm, w2bc_v)
    pltpu.sync_copy(bm2_hbm, bm2_v)
    pltpu.sync_copy(row_hbm.at[wid], ridx_v)
    pltpu.sync_copy(col_hbm.at[wid], cidx_v)

    row0 = sid * ROWS_PER_TILE
    cp_b = pltpu.async_copy(bm_hbm.at[pl.ds(row0, ROWS_PER_TILE)],
                            bm_sp.at[pl.ds(row0, ROWS_PER_TILE)], semscb)
    cp_z = pltpu.async_copy(z_hbm.at[pl.ds(row0, ROWS_PER_TILE)],
                            z_sp.at[pl.ds(row0, ROWS_PER_TILE)], semscb)
    for j in range(LANES):
        for jj in range(K // LANES):
            buf_v[0, j, pl.ds(jj * LANES, LANES)] = (
                jnp.zeros((LANES,), jnp.float32))
    zrows = [j * LANES for j in range(ROWS_PER_TILE // LANES)]
    zrows.append(ROWS_PER_TILE - LANES)
    for r in zrows:
        pltpu.async_copy(buf_v.at[0, pl.ds(0, LANES)],
                         acc_sp.at[pl.ds(row0 + r, LANES)], semsca)
    for r in zrows:
        pltpu.make_async_copy(buf_v.at[0, pl.ds(0, LANES)],
                              acc_sp.at[pl.ds(row0, LANES)], semsca).wait()
    cp_b.wait()
    cp_z.wait()
    plsc.subcore_barrier()

    bm2vec = bm2_v[...]
    iota = lax.iota(jnp.int32, LANES)
    zero16 = jnp.zeros((LANES,), jnp.int32)
    w2 = [w2bc_v[pl.ds(j * LANES, LANES)] for j in range(K // LANES)]
    ebase = wid * EDGES_PER_TILE

    def compute_chunk(t, ar_cur, br_cur, zr_cur, bbuf, bsem):
        base = ebase + t * CHUNK

        @plsc.parallel_loop(0, CHUNK, 1, unroll=8)
        def edge_body(e):
            acc = jnp.zeros((LANES,), jnp.float32)
            for j in range(K // LANES):
                va = ar_cur[e, pl.ds(j * LANES, LANES)]
                vb = br_cur[e, pl.ds(j * LANES, LANES)]
                hm = jnp.maximum(va + vb, 0.0)
                acc = acc + hm * w2[j]
            tv = jnp.broadcast_to(jnp.sum(acc, axis=0), (LANES,)) + bm2vec
            wv = 1.0 / (1.0 + jnp.exp(-tv))
            wrow_v[e, pl.ds(0, LANES)] = wv
            for j in range(K // LANES):
                vz = zr_cur[e, pl.ds(j * LANES, LANES)]
                bbuf[e, pl.ds(j * LANES, LANES)] = vz * wv

        for g in range(CHUNK // LANES):
            rows = g * LANES + iota
            w16 = plsc.load_gather(wrow_v, [rows, zero16])
            wbuf_v[pl.ds(g * LANES, LANES)] = w16
        pltpu.sync_copy(wbuf_v, w_hbm.at[pl.ds(base, CHUNK)])
        pltpu.async_copy(bbuf, acc_sp.at[ridx_v.at[t]], bsem, add=True)

    bufs = [(ar_v.at[0], br_v.at[0], zr_v.at[0], sem1a, sem2a, sem3a,
             buf_v.at[0], semsca),
            (ar_v.at[1], br_v.at[1], zr_v.at[1], sem1b, sem2b, sem3b,
             buf_v.at[1], semscb)]

    def issue(t, bs):
        pltpu.async_copy(am_hbm.at[ridx_v.at[t]], bs[0], bs[3])
        pltpu.async_copy(bm_sp.at[cidx_v.at[t]], bs[1], bs[4])
        pltpu.async_copy(z_sp.at[cidx_v.at[t]], bs[2], bs[5])

    def wait_for(t, bs):
        pltpu.make_async_copy(am_hbm.at[ridx_v.at[t]], bs[0], bs[3]).wait()
        pltpu.make_async_copy(bm_sp.at[cidx_v.at[t]], bs[1], bs[4]).wait()
        pltpu.make_async_copy(z_sp.at[cidx_v.at[t]], bs[2], bs[5]).wait()

    def wait_scatter(t, bs):
        pltpu.make_async_copy(bs[6], acc_sp.at[ridx_v.at[t]], bs[7]).wait()

    issue(0, bufs[0])

    def pair_body(u, carry):
        t0 = u * 2
        for b in range(2):
            t = t0 + b
            issue(t + 1, bufs[1 - b])
            wait_for(t, bufs[b])

            @pl.when(t >= 2)
            def _():
                wait_scatter(t, bufs[b])

            compute_chunk(t, bufs[b][0], bufs[b][1], bufs[b][2],
                          bufs[b][6], bufs[b][7])
        return carry

    lax.fori_loop(0, (NUM_CHUNKS - 1) // 2, pair_body, 0)
    t_last = NUM_CHUNKS - 1
    wait_for(t_last, bufs[0])
    wait_scatter(t_last, bufs[0])
    compute_chunk(t_last, bufs[0][0], bufs[0][1], bufs[0][2],
                  bufs[0][6], bufs[0][7])
    wait_scatter(t_last, bufs[0])
    wait_scatter(t_last, bufs[1])
    plsc.subcore_barrier()

    pltpu.sync_copy(acc_sp.at[pl.ds(row0, ROWS_PER_TILE)],
                    bz_hbm.at[cid, pl.ds(row0, ROWS_PER_TILE)])


def _sc_mlp_pass(am, bm, Z, row, col, w2bc, bm2_16):
    kern = pl.kernel(
        _mlp_edge_body,
        out_type=(
            jax.ShapeDtypeStruct((E,), jnp.float32),
            jax.ShapeDtypeStruct((NC, N, K), jnp.float32),
        ),
        mesh=_mesh(),
        compiler_params=pltpu.CompilerParams(needs_layout_passes=False, use_tc_tiling_on_sc=False),
        scratch_types=[
            pltpu.VMEM((NUM_CHUNKS, CHUNK), jnp.int32),
            pltpu.VMEM((NUM_CHUNKS, CHUNK), jnp.int32),
            pltpu.VMEM((2, CHUNK, K), jnp.float32),
            pltpu.VMEM((2, CHUNK, K), jnp.float32),
            pltpu.VMEM((2, CHUNK, K), jnp.float32),
            pltpu.VMEM((2, CHUNK, K), jnp.float32),
            pltpu.VMEM((CHUNK, 17), jnp.float32),
            pltpu.VMEM((CHUNK,), jnp.float32),
            pltpu.VMEM((K,), jnp.float32),
            pltpu.VMEM((LANES,), jnp.float32),
            pltpu.VMEM_SHARED((N, K), jnp.float32),
            pltpu.VMEM_SHARED((N, K), jnp.float32),
            pltpu.VMEM_SHARED((N, K), jnp.float32),
            pltpu.SemaphoreType.DMA,
            pltpu.SemaphoreType.DMA,
            pltpu.SemaphoreType.DMA,
            pltpu.SemaphoreType.DMA,
            pltpu.SemaphoreType.DMA,
            pltpu.SemaphoreType.DMA,
            pltpu.SemaphoreType.DMA,
            pltpu.SemaphoreType.DMA,
        ],
    )
    return kern(am, bm, Z, row, col, w2bc, bm2_16)


def kernel(X, ei_feat, ei_spatial, Wl1, bl1, Wr1, br1, att1, bias1,
           Wl2, bl2, Wr2, br2, att2, bias2, M, Wm1, bm1, Wm2, bm2):
    src = ei_feat[0].reshape(NC * NS, NUM_CHUNKS, CHUNK)
    dst = ei_feat[1].reshape(NC * NS, NUM_CHUNKS, CHUNK)
    row = ei_spatial[0].reshape(NC * NS, NUM_CHUNKS, CHUNK)
    col = ei_spatial[1].reshape(NC * NS, NUM_CHUNKS, CHUNK)

    Wcat = jnp.concatenate(
        [Wl1.T, Wr1.T, Wm1[:, :FD].T, Wm1[:, FD:].T], axis=1)
    bcat = jnp.concatenate(
        [bl1, br1, bm1, jnp.zeros_like(bm1)])[None, :]
    xl1, xr1, am, bm, S1 = _tc_k1(X, Wcat, bcat, att1[None, :])

    s1_16 = jnp.broadcast_to(jnp.reshape(S1, (1,)), (LANES,))
    P1 = _sc_gat_pass(H, xl1, xr1, src, dst, att1, s1_16)

    W2cat = jnp.concatenate([Wl2.T, Wr2.T], axis=1)
    b2cat = jnp.concatenate([bl2, br2])[None, :]
    xl2, xr2, S2 = _tc_k2(P1, xl1, xr1, att1[None, :], bias1[None, :], S1,
                          W2cat, b2cat, att2[None, :])

    s2_16 = jnp.broadcast_to(jnp.reshape(S2, (1,)), (LANES,))
    P2 = _sc_gat_pass(K, xl2, xr2, src, dst, att2, s2_16)

    Z = _tc_k3a(P2, xl2, xr2, att2[None, :], bias2[None, :], S2)

    bm2_16 = jnp.broadcast_to(bm2, (LANES,))
    w, BZ = _sc_mlp_pass(am, bm, Z, row, col, Wm2[0], bm2_16)

    out = _tc_k3b(Z, BZ, M)
    return (Z, out, w)


# speedup vs baseline: 1.0200x; 1.0200x over previous
import functools

import jax
import jax.numpy as jnp
from jax import lax
from jax.experimental import pallas as pl
from jax.experimental.pallas import tpu as pltpu
from jax.experimental.pallas import tpu_sc as plsc

N = 10000
FD = 128
H = 64
K = 32
E = 320000

NC = 2
NS = 16
LANES = 16

ROWS_PER_TILE = N // NS
CHUNK = 80
EDGES_PER_TILE = E // (NC * NS)
NUM_CHUNKS = EDGES_PER_TILE // CHUNK

_mesh = functools.partial(
    plsc.VectorSubcoreMesh, core_axis_name="c", subcore_axis_name="s",
    num_cores=NC, num_subcores=NS)


def _leaky(x):
    return jnp.where(x >= 0, x, 0.2 * x)


def _k1_body(x_ref, wcat_ref, bcat_ref, att1_ref,
             xl_ref, xr_ref, am_ref, bm_ref, s1_ref):
    x = x_ref[...]
    o = jnp.dot(x, wcat_ref[...], preferred_element_type=jnp.float32)
    o = o + bcat_ref[...]
    xl = o[:, :H]
    xr = o[:, H:2 * H]
    xl_ref[...] = xl
    xr_ref[...] = xr
    am_ref[...] = o[:, 2 * H:2 * H + K]
    bm_ref[...] = o[:, 2 * H + K:]
    colmax = jnp.max(jnp.abs(xl), axis=0) + jnp.max(jnp.abs(xr), axis=0)
    s1_ref[...] = jnp.sum(jnp.abs(att1_ref[0, :]) * colmax)[None, None]


def _tc_k1(X, Wcat, bcat, att1):
    return pl.pallas_call(
        _k1_body,
        out_shape=[
            jax.ShapeDtypeStruct((N, H), jnp.float32),
            jax.ShapeDtypeStruct((N, H), jnp.float32),
            jax.ShapeDtypeStruct((N, K), jnp.float32),
            jax.ShapeDtypeStruct((N, K), jnp.float32),
            jax.ShapeDtypeStruct((1, 1), jnp.float32),
        ],
    )(X, Wcat, bcat, att1)


def _k2_body(p_ref, xl1_ref, xr1_ref, att1_ref, bias1_ref, s1_ref,
             w2_ref, b2_ref, att2_ref,
             xl2_ref, xr2_ref, s2_ref):
    num = p_ref[0, :N, :H] + p_ref[1, :N, :H]
    dseg = p_ref[0, :N, H:H + 1] + p_ref[1, :N, H:H + 1]
    xl1 = xl1_ref[...]
    sl = jnp.sum(att1_ref[...] * _leaky(xl1 + xr1_ref[...]), axis=1,
                 keepdims=True)
    e_self = jnp.exp(sl - s1_ref[0, 0])
    den = dseg + e_self
    h = (num + e_self * xl1) / den + bias1_ref[...]
    h = jnp.where(h > 0, h, jnp.exp(jnp.minimum(h, 0.0)) - 1.0)
    hw = jnp.dot(h, w2_ref[...], preferred_element_type=jnp.float32)
    hw = hw + b2_ref[...]
    xl2 = hw[:, :K]
    xr2 = hw[:, K:]
    xl2_ref[...] = xl2
    xr2_ref[...] = xr2
    colmax = jnp.max(jnp.abs(xl2), axis=0) + jnp.max(jnp.abs(xr2), axis=0)
    s2_ref[...] = jnp.sum(jnp.abs(att2_ref[0, :]) * colmax)[None, None]


def _tc_k2(P1, xl1, xr1, att1, bias1, S1, W2cat, b2cat, att2):
    return pl.pallas_call(
        _k2_body,
        out_shape=[
            jax.ShapeDtypeStruct((N, K), jnp.float32),
            jax.ShapeDtypeStruct((N, K), jnp.float32),
            jax.ShapeDtypeStruct((1, 1), jnp.float32),
        ],
    )(P1, xl1, xr1, att1, bias1, S1, W2cat, b2cat, att2)


def _k3a_body(p_ref, xl2_ref, xr2_ref, att2_ref, bias2_ref, s2_ref, z_ref):
    num = p_ref[0, :N, :K] + p_ref[1, :N, :K]
    dseg = p_ref[0, :N, K:K + 1] + p_ref[1, :N, K:K + 1]
    xl2 = xl2_ref[...]
    sl = jnp.sum(att2_ref[...] * _leaky(xl2 + xr2_ref[...]), axis=1,
                 keepdims=True)
    e_self = jnp.exp(sl - s2_ref[0, 0])
    den = dseg + e_self
    o = (num + e_self * xl2) / den + bias2_ref[...]
    m = jnp.max(o, axis=1, keepdims=True)
    z = jnp.exp(o - m)
    z_ref[...] = z / jnp.sum(z, axis=1, keepdims=True)


def _tc_k3a(P2, xl2, xr2, att2, bias2, S2):
    return pl.pallas_call(
        _k3a_body,
        out_shape=jax.ShapeDtypeStruct((N, K), jnp.float32),
    )(P2, xl2, xr2, att2, bias2, S2)


def _k3b_body(z_ref, bz_ref, m_ref, out_ref):
    zt = z_ref[...] + bz_ref[0, :N, :] + bz_ref[1, :N, :]
    mr = jnp.maximum(m_ref[...], 0.0)
    out_ref[...] = jnp.dot(zt, mr, preferred_element_type=jnp.float32)


def _tc_k3b(Z, BZ, M):
    return pl.pallas_call(
        _k3b_body,
        out_shape=jax.ShapeDtypeStruct((N, FD), jnp.float32),
    )(Z, BZ, M)


def _gat_edge_body(D, W, STAGE,
                   xl_hbm, xr_hbm, src_hbm, dst_hbm, attbc_hbm, s_hbm,
                   p_hbm,
                   sidx_v, didx_v, xlr_v, xrr_v, buf_v, attbc_v, s_v,
                   xl_sp, xr_sp, acc_sp,
                   sem1a, sem2a, sem1b, sem2b, semsca, semscb):
    cid = lax.axis_index("c")
    sid = lax.axis_index("s")
    wid = cid * NS + sid

    pltpu.sync_copy(attbc_hbm, attbc_v)
    pltpu.sync_copy(s_hbm, s_v)
    pltpu.sync_copy(src_hbm.at[wid], sidx_v)
    pltpu.sync_copy(dst_hbm.at[wid], didx_v)

    row0 = sid * ROWS_PER_TILE
    if STAGE:
        cps = pltpu.async_copy(xl_hbm.at[pl.ds(row0, ROWS_PER_TILE)],
                               xl_sp.at[pl.ds(row0, ROWS_PER_TILE)], semscb)
        cpr = pltpu.async_copy(xr_hbm.at[pl.ds(row0, ROWS_PER_TILE)],
                               xr_sp.at[pl.ds(row0, ROWS_PER_TILE)], semscb)
    for j in range(LANES):
        for jj in range(W // LANES):
            buf_v[0, j, pl.ds(jj * LANES, LANES)] = (
                jnp.zeros((LANES,), jnp.float32))
    zrows = [j * LANES for j in range(ROWS_PER_TILE // LANES)]
    zrows.append(ROWS_PER_TILE - LANES)
    for r in zrows:
        pltpu.async_copy(buf_v.at[0, pl.ds(0, LANES)],
                         acc_sp.at[pl.ds(row0 + r, LANES)], semsca)
    for r in zrows:
        pltpu.make_async_copy(buf_v.at[0, pl.ds(0, LANES)],
                              acc_sp.at[pl.ds(row0, LANES)], semsca).wait()
    if STAGE:
        cps.wait()
        cpr.wait()
    plsc.subcore_barrier()

    svec = s_v[...]
    att = [attbc_v[pl.ds(j * LANES, LANES)] for j in range(D // LANES)]

    def compute_chunk(t, xl_cur, xr_cur, bbuf, bsem):
        @plsc.parallel_loop(0, CHUNK, 1, unroll=(4 if D > 32 else 8))
        def edge_body(e):
            acc1 = jnp.zeros((LANES,), jnp.float32)
            acc2 = jnp.zeros((LANES,), jnp.float32)
            vls = []
            for j in range(D // LANES):
                vl = xl_cur[e, pl.ds(j * LANES, LANES)]
                vr = xr_cur[e, pl.ds(j * LANES, LANES)]
                vls.append(vl)
                sj = vl + vr
                acc1 = acc1 + sj * att[j]
                acc2 = acc2 + jnp.abs(sj) * att[j]
            tot = 0.6 * acc1 + 0.4 * acc2
            ev = jnp.exp(
                jnp.broadcast_to(jnp.sum(tot, axis=0), (LANES,)) - svec)
            for j in range(D // LANES):
                bbuf[e, pl.ds(j * LANES, LANES)] = vls[j] * ev
            bbuf[e, pl.ds(D, LANES)] = ev

        pltpu.async_copy(bbuf, acc_sp.at[didx_v.at[t]], bsem, add=True)

    bufs = [(xlr_v.at[0], xrr_v.at[0], sem1a, sem2a, buf_v.at[0], semsca),
            (xlr_v.at[1], xrr_v.at[1], sem1b, sem2b, buf_v.at[1], semscb)]

    xl_src = xl_sp if STAGE else xl_hbm
    xr_src = xr_sp if STAGE else xr_hbm

    def issue(t, bs):
        pltpu.async_copy(xl_src.at[sidx_v.at[t]], bs[0], bs[2])
        pltpu.async_copy(xr_src.at[didx_v.at[t]], bs[1], bs[3])

    def wait_for(t, bs):
        pltpu.make_async_copy(xl_src.at[sidx_v.at[t]], bs[0], bs[2]).wait()
        pltpu.make_async_copy(xr_src.at[didx_v.at[t]], bs[1], bs[3]).wait()

    def wait_scatter(t, bs):
        pltpu.make_async_copy(bs[4], acc_sp.at[didx_v.at[t]], bs[5]).wait()

    issue(0, bufs[0])

    def pair_body(u, carry):
        t0 = u * 2
        for b in range(2):
            t = t0 + b
            issue(t + 1, bufs[1 - b])
            wait_for(t, bufs[b])

            @pl.when(t >= 2)
            def _():
                wait_scatter(t, bufs[b])

            compute_chunk(t, bufs[b][0], bufs[b][1], bufs[b][4], bufs[b][5])
        return carry

    lax.fori_loop(0, (NUM_CHUNKS - 1) // 2, pair_body, 0)
    t_last = NUM_CHUNKS - 1
    wait_for(t_last, bufs[0])
    wait_scatter(t_last, bufs[0])
    compute_chunk(t_last, bufs[0][0], bufs[0][1], bufs[0][4], bufs[0][5])
    wait_scatter(t_last, bufs[0])
    wait_scatter(t_last, bufs[1])
    plsc.subcore_barrier()

    pltpu.sync_copy(acc_sp.at[pl.ds(row0, ROWS_PER_TILE)],
                    p_hbm.at[cid, pl.ds(row0, ROWS_PER_TILE)])


def _sc_gat_pass(D, xl, xr, src, dst, attbc, s16):
    W = D + LANES
    stage = D <= 32
    body = functools.partial(_gat_edge_body, D, W, stage)
    kern = pl.kernel(
        body,
        out_type=jax.ShapeDtypeStruct((NC, N, W), jnp.float32),
        mesh=_mesh(),
        compiler_params=pltpu.CompilerParams(needs_layout_passes=False, use_tc_tiling_on_sc=False),
        scratch_types=[
            pltpu.VMEM((NUM_CHUNKS, CHUNK), jnp.int32),
            pltpu.VMEM((NUM_CHUNKS, CHUNK), jnp.int32),
            pltpu.VMEM((2, CHUNK, D), jnp.float32),
            pltpu.VMEM((2, CHUNK, D), jnp.float32),
            pltpu.VMEM((2, CHUNK, W), jnp.float32),
            pltpu.VMEM((D,), jnp.float32),
            pltpu.VMEM((LANES,), jnp.float32),
            pltpu.VMEM_SHARED((N, D) if stage else (8, LANES), jnp.float32),
            pltpu.VMEM_SHARED((N, D) if stage else (8, LANES), jnp.float32),
            pltpu.VMEM_SHARED((N, W), jnp.float32),
            pltpu.SemaphoreType.DMA,
            pltpu.SemaphoreType.DMA,
            pltpu.SemaphoreType.DMA,
            pltpu.SemaphoreType.DMA,
            pltpu.SemaphoreType.DMA,
            pltpu.SemaphoreType.DMA,
        ],
    )
    return kern(xl, xr, src, dst, attbc, s16)


def _mlp_edge_body(am_hbm, bm_hbm, z_hbm, row_hbm, col_hbm, w2bc_hbm, bm2_hbm,
                   w_hbm, bz_hbm,
                   ridx_v, cidx_v, ar_v, br_v, zr_v, buf_v, wrow_v, wbuf_v,
                   w2bc_v, bm2_v, bm_sp, z_sp, acc_sp,
                   sem1a, sem2a, sem3a, sem1b, sem2b, sem3b,
                   semsca, semscb):
    cid = lax.axis_index("c")
    sid = lax.axis_index("s")
    wid = cid * NS + sid

    pltpu.sync_copy(w2bc_hbm, w2bc_v)
    pltpu.sync_copy(bm2_hbm, bm2_v)
    pltpu.sync_copy(row_hbm.at[wid], ridx_v)
    pltpu.sync_copy(col_hbm.at[wid], cidx_v)

    row0 = sid * ROWS_PER_TILE
    cp_b = pltpu.async_copy(bm_hbm.at[pl.ds(row0, ROWS_PER_TILE)],
                            bm_sp.at[pl.ds(row0, ROWS_PER_TILE)], semscb)
    cp_z = pltpu.async_copy(z_hbm.at[pl.ds(row0, ROWS_PER_TILE)],
                            z_sp.at[pl.ds(row0, ROWS_PER_TILE)], semscb)
    for j in range(LANES):
        for jj in range(K // LANES):
            buf_v[0, j, pl.ds(jj * LANES, LANES)] = (
                jnp.zeros((LANES,), jnp.float32))
    zrows = [j * LANES for j in range(ROWS_PER_TILE // LANES)]
    zrows.append(ROWS_PER_TILE - LANES)
    for r in zrows:
        pltpu.async_copy(buf_v.at[0, pl.ds(0, LANES)],
                         acc_sp.at[pl.ds(row0 + r, LANES)], semsca)
    for r in zrows:
        pltpu.make_async_copy(buf_v.at[0, pl.ds(0, LANES)],
                              acc_sp.at[pl.ds(row0, LANES)], semsca).wait()
    cp_b.wait()
    cp_z.wait()
    plsc.subcore_barrier()

    bm2vec = bm2_v[...]
    iota = lax.iota(jnp.int32, LANES)
    zero16 = jnp.zeros((LANES,), jnp.int32)
    w2 = [w2bc_v[pl.ds(j * LANES, LANES)] for j in range(K // LANES)]
    ebase = wid * EDGES_PER_TILE

    def compute_chunk(t, ar_cur, br_cur, zr_cur, bbuf, bsem):
        base = ebase + t * CHUNK

        @plsc.parallel_loop(0, CHUNK, 1, unroll=8)
        def edge_body(e):
            acc = jnp.zeros((LANES,), jnp.float32)
            for j in range(K // LANES):
                va = ar_cur[e, pl.ds(j * LANES, LANES)]
                vb = br_cur[e, pl.ds(j * LANES, LANES)]
                hm = jnp.maximum(va + vb, 0.0)
                acc = acc + hm * w2[j]
            tv = jnp.broadcast_to(jnp.sum(acc, axis=0), (LANES,)) + bm2vec
            wv = 1.0 / (1.0 + jnp.exp(-tv))
            wrow_v[e, pl.ds(0, LANES)] = wv
            for j in range(K // LANES):
                vz = zr_cur[e, pl.ds(j * LANES, LANES)]
                bbuf[e, pl.ds(j * LANES, LANES)] = vz * wv

        for g in range(CHUNK // LANES):
            rows = g * LANES + iota
            w16 = plsc.load_gather(wrow_v, [rows, zero16])
            wbuf_v[pl.ds(g * LANES, LANES)] = w16
        pltpu.sync_copy(wbuf_v, w_hbm.at[pl.ds(base, CHUNK)])
        pltpu.async_copy(bbuf, acc_sp.at[ridx_v.at[t]], bsem, add=True)

    bufs = [(ar_v.at[0], br_v.at[0], zr_v.at[0], sem1a, sem2a, sem3a,
             buf_v.at[0], semsca),
            (ar_v.at[1], br_v.at[1], zr_v.at[1], sem1b, sem2b, sem3b,
             buf_v.at[1], semscb)]

    def issue(t, bs):
        pltpu.async_copy(am_hbm.at[ridx_v.at[t]], bs[0], bs[3])
        pltpu.async_copy(bm_sp.at[cidx_v.at[t]], bs[1], bs[4])
        pltpu.async_copy(z_sp.at[cidx_v.at[t]], bs[2], bs[5])

    def wait_for(t, bs):
        pltpu.make_async_copy(am_hbm.at[ridx_v.at[t]], bs[0], bs[3]).wait()
        pltpu.make_async_copy(bm_sp.at[cidx_v.at[t]], bs[1], bs[4]).wait()
        pltpu.make_async_copy(z_sp.at[cidx_v.at[t]], bs[2], bs[5]).wait()

    def wait_scatter(t, bs):
        pltpu.make_async_copy(bs[6], acc_sp.at[ridx_v.at[t]], bs[7]).wait()

    issue(0, bufs[0])

    def pair_body(u, carry):
        t0 = u * 2
        for b in range(2):
            t = t0 + b
            issue(t + 1, bufs[1 - b])
            wait_for(t, bufs[b])

            @pl.when(t >= 2)
            def _():
                wait_scatter(t, bufs[b])

            compute_chunk(t, bufs[b][0], bufs[b][1], bufs[b][2],
                          bufs[b][6], bufs[b][7])
        return carry

    lax.fori_loop(0, (NUM_CHUNKS - 1) // 2, pair_body, 0)
    t_last = NUM_CHUNKS - 1
    wait_for(t_last, bufs[0])
    wait_scatter(t_last, bufs[0])
    compute_chunk(t_last, bufs[0][0], bufs[0][1], bufs[0][2],
                  bufs[0][6], bufs[0][7])
    wait_scatter(t_last, bufs[0])
    wait_scatter(t_last, bufs[1])
    plsc.subcore_barrier()

    pltpu.sync_copy(acc_sp.at[pl.ds(row0, ROWS_PER_TILE)],
                    bz_hbm.at[cid, pl.ds(row0, ROWS_PER_TILE)])


def _sc_mlp_pass(am, bm, Z, row, col, w2bc, bm2_16):
    kern = pl.kernel(
        _mlp_edge_body,
        out_type=(
            jax.ShapeDtypeStruct((E,), jnp.float32),
            jax.ShapeDtypeStruct((NC, N, K), jnp.float32),
        ),
        mesh=_mesh(),
        compiler_params=pltpu.CompilerParams(needs_layout_passes=False, use_tc_tiling_on_sc=False),
        scratch_types=[
            pltpu.VMEM((NUM_CHUNKS, CHUNK), jnp.int32),
            pltpu.VMEM((NUM_CHUNKS, CHUNK), jnp.int32),
            pltpu.VMEM((2, CHUNK, K), jnp.float32),
            pltpu.VMEM((2, CHUNK, K), jnp.float32),
            pltpu.VMEM((2, CHUNK, K), jnp.float32),
            pltpu.VMEM((2, CHUNK, K), jnp.float32),
            pltpu.VMEM((CHUNK, 17), jnp.float32),
            pltpu.VMEM((CHUNK,), jnp.float32),
            pltpu.VMEM((K,), jnp.float32),
            pltpu.VMEM((LANES,), jnp.float32),
            pltpu.VMEM_SHARED((N, K), jnp.float32),
            pltpu.VMEM_SHARED((N, K), jnp.float32),
            pltpu.VMEM_SHARED((N, K), jnp.float32),
            pltpu.SemaphoreType.DMA,
            pltpu.SemaphoreType.DMA,
            pltpu.SemaphoreType.DMA,
            pltpu.SemaphoreType.DMA,
            pltpu.SemaphoreType.DMA,
            pltpu.SemaphoreType.DMA,
            pltpu.SemaphoreType.DMA,
            pltpu.SemaphoreType.DMA,
        ],
    )
    return kern(am, bm, Z, row, col, w2bc, bm2_16)


def kernel(X, ei_feat, ei_spatial, Wl1, bl1, Wr1, br1, att1, bias1,
           Wl2, bl2, Wr2, br2, att2, bias2, M, Wm1, bm1, Wm2, bm2):
    src = ei_feat[0].reshape(NC * NS, NUM_CHUNKS, CHUNK)
    dst = ei_feat[1].reshape(NC * NS, NUM_CHUNKS, CHUNK)
    row = ei_spatial[0].reshape(NC * NS, NUM_CHUNKS, CHUNK)
    col = ei_spatial[1].reshape(NC * NS, NUM_CHUNKS, CHUNK)

    Wcat = jnp.concatenate(
        [Wl1.T, Wr1.T, Wm1[:, :FD].T, Wm1[:, FD:].T], axis=1)
    bcat = jnp.concatenate(
        [bl1, br1, bm1, jnp.zeros_like(bm1)])[None, :]
    xl1, xr1, am, bm, S1 = _tc_k1(X, Wcat, bcat, att1[None, :])

    s1_16 = jnp.broadcast_to(jnp.reshape(S1, (1,)), (LANES,))
    P1 = _sc_gat_pass(H, xl1, xr1, src, dst, att1, s1_16)

    W2cat = jnp.concatenate([Wl2.T, Wr2.T], axis=1)
    b2cat = jnp.concatenate([bl2, br2])[None, :]
    xl2, xr2, S2 = _tc_k2(P1, xl1, xr1, att1[None, :], bias1[None, :], S1,
                          W2cat, b2cat, att2[None, :])

    s2_16 = jnp.broadcast_to(jnp.reshape(S2, (1,)), (LANES,))
    P2 = _sc_gat_pass(K, xl2, xr2, src, dst, att2, s2_16)

    Z = _tc_k3a(P2, xl2, xr2, att2[None, :], bias2[None, :], S2)

    bm2_16 = jnp.broadcast_to(bm2, (LANES,))
    w, BZ = _sc_mlp_pass(am, bm, Z, row, col, Wm2[0], bm2_16)

    out = _tc_k3b(Z, BZ, M)
    return (Z, out, w)
